# baseline (speedup 1.0000x reference)
import functools

import jax
import jax.numpy as jnp
from jax import lax
from jax.experimental import pallas as pl
from jax.experimental.pallas import tpu as pltpu
from jax.experimental.pallas import tpu_sc as plsc

NUM_CORES = 2
NUM_SUBCORES = 16
NUM_TILES = NUM_CORES * NUM_SUBCORES
LANES = 16
CHUNK = 128
N_COMB = 13 * 32


def _comb_body(m_ref, d_ref, o_ref):
    m = m_ref[...]
    d = d_ref[...]
    o_ref[...] = (m[:, None, :] + d[None, :, :]).reshape(N_COMB, 128)


def _make_comb(month_table, day_table):
    return pl.pallas_call(
        _comb_body,
        out_shape=jax.ShapeDtypeStruct((N_COMB, 128), jnp.float32),
    )(month_table, day_table)


def _pack_bf16_pairs(table):
    v, h = table.shape
    bf = table.astype(jnp.bfloat16).reshape(v, h // 2, 2)
    return jax.lax.bitcast_convert_type(bf, jnp.int32)


def _make_sc_kernel(n, hidden):
    per_tile = n // NUM_TILES
    n_chunks = per_tile // CHUNK
    assert n_chunks % 2 == 0 and n_chunks >= 6
    packed = hidden // 2
    mesh = plsc.VectorSubcoreMesh(
        core_axis_name="c", subcore_axis_name="s",
        num_cores=NUM_CORES, num_subcores=NUM_SUBCORES,
    )

    idx_t = pltpu.VMEM((CHUNK,), jnp.int32)
    rows_i_t = pltpu.VMEM((CHUNK, packed), jnp.int32)
    rows_f_t = pltpu.VMEM((CHUNK, hidden), jnp.float32)

    @functools.partial(
        pl.kernel,
        out_type=jax.ShapeDtypeStruct((n, hidden), jnp.float32),
        mesh=mesh,
        scratch_types=[idx_t] * 8 + [rows_i_t] * 2 + [rows_f_t] * 2
        + [pltpu.VMEM((N_COMB * packed,), jnp.int32)]
        + [pltpu.SemaphoreType.DMA] * 6,
        compiler_params=pltpu.CompilerParams(use_tc_tiling_on_sc=False,
                                             needs_layout_passes=False),
    )
    def _sc(year_hbm, month_hbm, day_hbm, ytab_hbm, ctab_hbm, out_hbm,
            yi0, yi1, mi0, mi1, di0, di1, md0, md1,
            ry0, ry1, ro0, ro1, comb_v,
            semi0, semi1, semg0, semg1, semo0, semo1):
        yidx, midx, didx, md = (yi0, yi1), (mi0, mi1), (di0, di1), (md0, md1)
        rows_y, rows_o = (ry0, ry1), (ro0, ro1)
        sem_idx, sem_g, sem_out = (semi0, semi1), (semg0, semg1), (semo0, semo1)

        wid = lax.axis_index("s") * NUM_CORES + lax.axis_index("c")
        base0 = wid * per_tile

        lane = lax.iota(jnp.int32, LANES)

        pltpu.sync_copy(ctab_hbm, comb_v)

        def issue_idx(c, b):
            base = base0 + c * CHUNK
            pltpu.async_copy(year_hbm.at[pl.ds(base, CHUNK)], yidx[b], sem_idx[b])
            pltpu.async_copy(month_hbm.at[pl.ds(base, CHUNK)], midx[b], sem_idx[b])
            pltpu.async_copy(day_hbm.at[pl.ds(base, CHUNK)], didx[b], sem_idx[b])

        def wait_idx(b):
            pltpu.make_async_copy(year_hbm.at[pl.ds(0, CHUNK)], yidx[b], sem_idx[b]).wait()
            pltpu.make_async_copy(month_hbm.at[pl.ds(0, CHUNK)], midx[b], sem_idx[b]).wait()
            pltpu.make_async_copy(day_hbm.at[pl.ds(0, CHUNK)], didx[b], sem_idx[b]).wait()

        def compute_md(b):
            for k in range(CHUNK // LANES):
                s = pl.ds(k * LANES, LANES)
                md[b][s] = midx[b][s] * 32 + didx[b][s]

        def issue_gather(b):
            pltpu.async_copy(ytab_hbm.at[yidx[b]], rows_y[b], sem_g[b])

        def wait_gather(b):
            pltpu.make_async_copy(ytab_hbm.at[yidx[b]], rows_y[b], sem_g[b]).wait()

        def issue_out(c, b):
            base = base0 + c * CHUNK
            pltpu.async_copy(rows_o[b], out_hbm.at[pl.ds(base, CHUNK)], sem_out[b])

        def wait_out(b):
            pltpu.make_async_copy(rows_o[b], out_hbm.at[pl.ds(0, CHUNK)], sem_out[b]).wait()

        def add_rows(b):
            n_blk = CHUNK // LANES
            mdv = [md[b][pl.ds(p * LANES, LANES)] for p in range(n_blk)]
            cbase = [v * packed for v in mdv]
            rowv = [lane + p * LANES for p in range(n_blk)]

            @pl.loop(0, packed)
            def _col_loop(jp):
                jps = jnp.full((LANES,), jp, jnp.int32)
                col_e = jps * 2
                col_o = col_e + 1
                for p in range(n_blk):
                    yv = plsc.load_gather(rows_y[b], [rowv[p], jps])
                    cv = plsc.load_gather(comb_v, [cbase[p] + jp])
                    y_e = plsc.bitcast(yv << 16, jnp.float32)
                    y_o = plsc.bitcast(yv & -65536, jnp.float32)
                    c_e = plsc.bitcast(cv << 16, jnp.float32)
                    c_o = plsc.bitcast(cv & -65536, jnp.float32)
                    plsc.store_scatter(rows_o[b], [rowv[p], col_e], y_e + c_e)
                    plsc.store_scatter(rows_o[b], [rowv[p], col_o], y_o + c_o)

        def step(c, b, wait_prev_out, next_gather, next_idx):
            nb = 1 - b
            if next_gather:
                wait_idx(nb)
                compute_md(nb)
                issue_gather(nb)
            wait_gather(b)
            if next_idx:
                issue_idx(c + 2, b)
            if wait_prev_out:
                wait_out(b)
            add_rows(b)
            issue_out(c, b)

        issue_idx(0, 0)
        issue_idx(1, 1)
        wait_idx(0)
        compute_md(0)
        issue_gather(0)
        step(0, 0, False, True, True)
        step(1, 1, False, True, True)

        @pl.loop(2, n_chunks - 2, step=2)
        def _main(c):
            step(c, 0, True, True, True)
            step(c + 1, 1, True, True, True)

        step(n_chunks - 2, 0, True, True, False)
        step(n_chunks - 1, 1, True, False, False)
        wait_out(0)
        wait_out(1)

    return _sc


def kernel(year, month, day, year_table, month_table, day_table):
    b, l = year.shape
    hidden = year_table.shape[1]
    n = b * l
    yidx = year.reshape(n).astype(jnp.int32)
    midx = month.reshape(n).astype(jnp.int32)
    didx = day.reshape(n).astype(jnp.int32)
    ytab_i32 = _pack_bf16_pairs(year_table.astype(jnp.float32))
    comb = _pack_bf16_pairs(_make_comb(month_table.astype(jnp.float32),
                                       day_table.astype(jnp.float32)))
    comb_flat = comb.reshape(N_COMB * (hidden // 2))
    sc = _make_sc_kernel(n, hidden)
    out = sc(yidx, midx, didx, ytab_i32, comb_flat)
    return out.reshape(b, l, hidden)


# speedup vs baseline: 5.2532x; 5.2532x over previous
import functools

import jax
import jax.numpy as jnp
from jax import lax
from jax.experimental import pallas as pl
from jax.experimental.pallas import tpu as pltpu
from jax.experimental.pallas import tpu_sc as plsc

NUM_CORES = 2
NUM_SUBCORES = 16
NUM_TILES = NUM_CORES * NUM_SUBCORES
LANES = 16
CHUNK = 128
N_COMB = 13 * 32


def _comb_body(m_ref, d_ref, o_ref):
    m = m_ref[...]
    d = d_ref[...]
    o_ref[...] = (m[:, None, :] + d[None, :, :]).reshape(N_COMB, 128)


def _make_comb(month_table, day_table):
    return pl.pallas_call(
        _comb_body,
        out_shape=jax.ShapeDtypeStruct((N_COMB, 128), jnp.float32),
    )(month_table, day_table)


def _pack_bf16_pairs(table):
    v, h = table.shape
    bf = table.astype(jnp.bfloat16).reshape(v, h // 32, 2, LANES)
    bf = bf.transpose(0, 1, 3, 2)
    return jax.lax.bitcast_convert_type(bf, jnp.int32).reshape(v, h // 2)


def _make_sc_kernel(n, hidden):
    per_tile = n // NUM_TILES
    n_chunks = per_tile // CHUNK
    assert n_chunks % 2 == 0 and n_chunks >= 6
    packed = hidden // 2
    mesh = plsc.VectorSubcoreMesh(
        core_axis_name="c", subcore_axis_name="s",
        num_cores=NUM_CORES, num_subcores=NUM_SUBCORES,
    )

    idx_t = pltpu.VMEM((CHUNK,), jnp.int32)
    rows_i_t = pltpu.VMEM((CHUNK, packed), jnp.int32)
    rows_f_t = pltpu.VMEM((CHUNK, hidden), jnp.float32)

    @functools.partial(
        pl.kernel,
        out_type=jax.ShapeDtypeStruct((n, hidden), jnp.float32),
        mesh=mesh,
        scratch_types=[idx_t] * 8 + [rows_i_t] * 4 + [rows_f_t] * 2
        + [pltpu.SemaphoreType.DMA] * 6,
        compiler_params=pltpu.CompilerParams(use_tc_tiling_on_sc=False,
                                             needs_layout_passes=False),
    )
    def _sc(year_hbm, month_hbm, day_hbm, ytab_hbm, ctab_hbm, out_hbm,
            yi0, yi1, mi0, mi1, di0, di1, md0, md1,
            ry0, ry1, rc0, rc1, ro0, ro1,
            semi0, semi1, semg0, semg1, semo0, semo1):
        yidx, midx, didx, md = (yi0, yi1), (mi0, mi1), (di0, di1), (md0, md1)
        rows_y, rows_c, rows_o = (ry0, ry1), (rc0, rc1), (ro0, ro1)
        sem_idx, sem_g, sem_out = (semi0, semi1), (semg0, semg1), (semo0, semo1)

        wid = lax.axis_index("s") * NUM_CORES + lax.axis_index("c")
        base0 = wid * per_tile

        def issue_idx(c, b):
            base = base0 + c * CHUNK
            pltpu.async_copy(year_hbm.at[pl.ds(base, CHUNK)], yidx[b], sem_idx[b])
            pltpu.async_copy(month_hbm.at[pl.ds(base, CHUNK)], midx[b], sem_idx[b])
            pltpu.async_copy(day_hbm.at[pl.ds(base, CHUNK)], didx[b], sem_idx[b])

        def wait_idx(b):
            pltpu.make_async_copy(year_hbm.at[pl.ds(0, CHUNK)], yidx[b], sem_idx[b]).wait()
            pltpu.make_async_copy(month_hbm.at[pl.ds(0, CHUNK)], midx[b], sem_idx[b]).wait()
            pltpu.make_async_copy(day_hbm.at[pl.ds(0, CHUNK)], didx[b], sem_idx[b]).wait()

        def compute_md(b):
            for k in range(CHUNK // LANES):
                s = pl.ds(k * LANES, LANES)
                md[b][s] = midx[b][s] * 32 + didx[b][s]

        def issue_gather(b):
            pltpu.async_copy(ytab_hbm.at[yidx[b]], rows_y[b], sem_g[b])
            pltpu.async_copy(ctab_hbm.at[md[b]], rows_c[b], sem_g[b])

        def wait_gather(b):
            pltpu.make_async_copy(ytab_hbm.at[yidx[b]], rows_y[b], sem_g[b]).wait()
            pltpu.make_async_copy(ctab_hbm.at[md[b]], rows_c[b], sem_g[b]).wait()

        def issue_out(c, b):
            base = base0 + c * CHUNK
            pltpu.async_copy(rows_o[b], out_hbm.at[pl.ds(base, CHUNK)], sem_out[b])

        def wait_out(b):
            pltpu.make_async_copy(rows_o[b], out_hbm.at[pl.ds(0, CHUNK)], sem_out[b]).wait()

        def add_rows(b):
            @pl.loop(0, CHUNK)
            def _row_loop(i):
                for g in range(hidden // 32):
                    s = pl.ds(LANES * g, LANES)
                    ybits = rows_y[b][i, s]
                    cbits = rows_c[b][i, s]
                    y_lo = plsc.bitcast(ybits << 16, jnp.float32)
                    y_hi = plsc.bitcast(ybits & -65536, jnp.float32)
                    c_lo = plsc.bitcast(cbits << 16, jnp.float32)
                    c_hi = plsc.bitcast(cbits & -65536, jnp.float32)
                    rows_o[b][i, pl.ds(32 * g, LANES)] = y_lo + c_lo
                    rows_o[b][i, pl.ds(32 * g + LANES, LANES)] = y_hi + c_hi

        def step(c, b, wait_prev_out, next_gather, next_idx):
            nb = 1 - b
            if next_gather:
                wait_idx(nb)
                compute_md(nb)
                issue_gather(nb)
            wait_gather(b)
            if next_idx:
                issue_idx(c + 2, b)
            if wait_prev_out:
                wait_out(b)
            add_rows(b)
            issue_out(c, b)

        issue_idx(0, 0)
        issue_idx(1, 1)
        wait_idx(0)
        compute_md(0)
        issue_gather(0)
        step(0, 0, False, True, True)
        step(1, 1, False, True, True)

        @pl.loop(2, n_chunks - 2, step=2)
        def _main(c):
            step(c, 0, True, True, True)
            step(c + 1, 1, True, True, True)

        step(n_chunks - 2, 0, True, True, False)
        step(n_chunks - 1, 1, True, False, False)
        wait_out(0)
        wait_out(1)

    return _sc


def kernel(year, month, day, year_table, month_table, day_table):
    b, l = year.shape
    hidden = year_table.shape[1]
    n = b * l
    yidx = year.reshape(n).astype(jnp.int32)
    midx = month.reshape(n).astype(jnp.int32)
    didx = day.reshape(n).astype(jnp.int32)
    ytab_i32 = _pack_bf16_pairs(year_table.astype(jnp.float32))
    comb = _pack_bf16_pairs(_make_comb(month_table.astype(jnp.float32),
                                       day_table.astype(jnp.float32)))
    sc = _make_sc_kernel(n, hidden)
    out = sc(yidx, midx, didx, ytab_i32, comb)
    return out.reshape(b, l, hidden)


# speedup vs baseline: 5.5037x; 1.0477x over previous
import functools

import jax
import jax.numpy as jnp
from jax import lax
from jax.experimental import pallas as pl
from jax.experimental.pallas import tpu as pltpu
from jax.experimental.pallas import tpu_sc as plsc

NUM_CORES = 2
NUM_SUBCORES = 16
NUM_TILES = NUM_CORES * NUM_SUBCORES
LANES = 16
CHUNK = 128


def _comb_body(m_ref, d_ref, o_ref):
    m = m_ref[...]
    d = d_ref[...]
    o_ref[...] = (m[:, None, :] + d[None, :, :]).reshape(13 * 32, 128)


def _make_comb(month_table, day_table):
    return pl.pallas_call(
        _comb_body,
        out_shape=jax.ShapeDtypeStruct((13 * 32, 128), jnp.float32),
    )(month_table, day_table)


def _make_sc_kernel(n, hidden):
    per_tile = n // NUM_TILES
    n_chunks = per_tile // CHUNK
    assert n_chunks % 2 == 0 and n_chunks >= 6
    mesh = plsc.VectorSubcoreMesh(
        core_axis_name="c", subcore_axis_name="s",
        num_cores=NUM_CORES, num_subcores=NUM_SUBCORES,
    )

    idx_t = pltpu.VMEM((CHUNK,), jnp.int32)
    rows_t = pltpu.VMEM((CHUNK, hidden), jnp.float32)

    @functools.partial(
        pl.kernel,
        out_type=jax.ShapeDtypeStruct((n, hidden), jnp.float32),
        mesh=mesh,
        scratch_types=[idx_t] * 8 + [rows_t] * 4
        + [pltpu.SemaphoreType.DMA] * 6,
    )
    def _sc(year_hbm, month_hbm, day_hbm, ytab_hbm, ctab_hbm, out_hbm,
            yi0, yi1, mi0, mi1, di0, di1, md0, md1,
            ry0, ry1, rc0, rc1,
            semi0, semi1, semg0, semg1, semo0, semo1):
        yidx, midx, didx, md = (yi0, yi1), (mi0, mi1), (di0, di1), (md0, md1)
        rows_y, rows_c = (ry0, ry1), (rc0, rc1)
        sem_idx, sem_g, sem_out = (semi0, semi1), (semg0, semg1), (semo0, semo1)

        wid = lax.axis_index("s") * NUM_CORES + lax.axis_index("c")
        base0 = wid * per_tile

        def issue_idx(c, b):
            base = base0 + c * CHUNK
            pltpu.async_copy(year_hbm.at[pl.ds(base, CHUNK)], yidx[b], sem_idx[b])
            pltpu.async_copy(month_hbm.at[pl.ds(base, CHUNK)], midx[b], sem_idx[b])
            pltpu.async_copy(day_hbm.at[pl.ds(base, CHUNK)], didx[b], sem_idx[b])

        def wait_idx(b):
            pltpu.make_async_copy(year_hbm.at[pl.ds(0, CHUNK)], yidx[b], sem_idx[b]).wait()
            pltpu.make_async_copy(month_hbm.at[pl.ds(0, CHUNK)], midx[b], sem_idx[b]).wait()
            pltpu.make_async_copy(day_hbm.at[pl.ds(0, CHUNK)], didx[b], sem_idx[b]).wait()

        def compute_md(b):
            for k in range(CHUNK // LANES):
                s = pl.ds(k * LANES, LANES)
                md[b][s] = midx[b][s] * 32 + didx[b][s]

        def issue_gather(b):
            pltpu.async_copy(ytab_hbm.at[yidx[b]], rows_y[b], sem_g[b])
            pltpu.async_copy(ctab_hbm.at[md[b]], rows_c[b], sem_g[b])

        def wait_gather(b):
            pltpu.make_async_copy(ytab_hbm.at[yidx[b]], rows_y[b], sem_g[b]).wait()
            pltpu.make_async_copy(ctab_hbm.at[md[b]], rows_c[b], sem_g[b]).wait()

        def issue_out(c, b):
            base = base0 + c * CHUNK
            pltpu.async_copy(rows_y[b], out_hbm.at[pl.ds(base, CHUNK)], sem_out[b])

        def wait_out(b):
            pltpu.make_async_copy(rows_y[b], out_hbm.at[pl.ds(0, CHUNK)], sem_out[b]).wait()

        def add_rows(b):
            @pl.loop(0, CHUNK)
            def _row_loop(i):
                for k in range(hidden // LANES):
                    s = pl.ds(k * LANES, LANES)
                    plsc.addupdate(rows_y[b].at[i, s], rows_c[b][i, s])

        def step(c, b, wait_prev_out, next_gather, next_idx):
            nb = 1 - b
            if next_gather:
                wait_idx(nb)
                compute_md(nb)
                if wait_prev_out:
                    wait_out(nb)
                issue_gather(nb)
            wait_gather(b)
            if next_idx:
                issue_idx(c + 2, b)
            add_rows(b)
            issue_out(c, b)

        issue_idx(0, 0)
        issue_idx(1, 1)
        wait_idx(0)
        compute_md(0)
        issue_gather(0)
        step(0, 0, False, True, True)

        @pl.loop(1, n_chunks - 3, step=2)
        def _main(c):
            step(c, 1, True, True, True)
            step(c + 1, 0, True, True, True)

        step(n_chunks - 3, 1, True, True, True)
        step(n_chunks - 2, 0, True, True, False)
        step(n_chunks - 1, 1, True, False, False)
        wait_out(0)
        wait_out(1)

    return _sc


def kernel(year, month, day, year_table, month_table, day_table):
    b, l = year.shape
    hidden = year_table.shape[1]
    n = b * l
    yidx = year.reshape(n).astype(jnp.int32)
    midx = month.reshape(n).astype(jnp.int32)
    didx = day.reshape(n).astype(jnp.int32)
    comb = _make_comb(month_table.astype(jnp.float32),
                      day_table.astype(jnp.float32))
    sc = _make_sc_kernel(n, hidden)
    out = sc(yidx, midx, didx, year_table, comb)
    return out.reshape(b, l, hidden)
